# baseline (speedup 1.0000x reference)
import functools

import jax
import jax.numpy as jnp
from jax import lax
from jax.experimental import pallas as pl
from jax.experimental.pallas import tpu as pltpu
from jax.experimental.pallas import tpu_sc as plsc

B = 4096
L = 50
D = 32
NC = 2
NS = 16
NW = NC * NS
BPW = B // NW
G = 2
IPG = G * L
IPG_PAD = 104
NG = BPW // G
NBUF = 4
INV_L = 1.0 / L

_mesh = plsc.VectorSubcoreMesh(core_axis_name="c", subcore_axis_name="s")


@functools.partial(
    pl.kernel,
    out_type=jax.ShapeDtypeStruct((B, D), jnp.float32),
    mesh=_mesh,
    compiler_params=pltpu.CompilerParams(use_tc_tiling_on_sc=False),
    scratch_types=[
        pltpu.VMEM((NG * IPG_PAD,), jnp.int32),
        pltpu.VMEM((NBUF, IPG_PAD, D), jnp.float32),
        pltpu.VMEM((BPW, D), jnp.float32),
        [pltpu.SemaphoreType.DMA] * NBUF,
    ],
)
def _avg_embed(idx_hbm, table_hbm, out_hbm, idx_v, rows_v, out_v, sems):
    wid = lax.axis_index("s") * NC + lax.axis_index("c")
    base = wid * BPW

    pltpu.sync_copy(idx_hbm.at[pl.ds(wid * NG * IPG_PAD, NG * IPG_PAD)], idx_v)

    def fire(g, slot):
        return pltpu.async_copy(
            table_hbm.at[idx_v.at[pl.ds(g * IPG_PAD, IPG_PAD)]],
            rows_v.at[slot],
            sems[slot],
        )

    def drain(slot):
        pltpu.make_async_copy(
            table_hbm.at[idx_v.at[pl.ds(0, IPG_PAD)]],
            rows_v.at[slot],
            sems[slot],
        ).wait()

    for b in range(NBUF):
        fire(b, b)

    def outer(gg):
        for b in range(NBUF):
            g = gg + b
            drain(b)

            @pl.when(g + NBUF < NG)
            def _():
                fire(g + NBUF, b)

            buf = rows_v.at[b]
            for j in range(G):
                acc0 = jnp.zeros((16,), jnp.float32)
                acc1 = jnp.zeros((16,), jnp.float32)
                for l in range(L):
                    r = j * L + l
                    acc0 = acc0 + buf[r, pl.ds(0, 16)]
                    acc1 = acc1 + buf[r, pl.ds(16, 16)]
                row = g * G + j
                out_v[row, pl.ds(0, 16)] = acc0 * INV_L
                out_v[row, pl.ds(16, 16)] = acc1 * INV_L

    pl.loop(0, NG, step=NBUF)(outer)

    pltpu.sync_copy(out_v, out_hbm.at[pl.ds(base, BPW), :])


def kernel(inputs, table):
    idx = inputs.reshape(B // G, G * L)
    idx = jnp.pad(idx, ((0, 0), (0, IPG_PAD - IPG)))
    return _avg_embed(idx.reshape(-1), table)


# speedup vs baseline: 1.5855x; 1.5855x over previous
import functools

import jax
import jax.numpy as jnp
from jax import lax
from jax.experimental import pallas as pl
from jax.experimental.pallas import tpu as pltpu
from jax.experimental.pallas import tpu_sc as plsc

B = 4096
L = 50
D = 32
NC = 2
NS = 16
NW = NC * NS
BPW = B // NW
CROWS = 16
NCHUNK = BPW // CROWS
NBUF = 2
INV_L = 1.0 / L

_mesh = plsc.VectorSubcoreMesh(core_axis_name="c", subcore_axis_name="s")


@functools.partial(
    pl.kernel,
    out_type=jax.ShapeDtypeStruct((B, D), jnp.float32),
    mesh=_mesh,
    scratch_types=[
        pltpu.VMEM((NBUF, L, D), jnp.float32),
        pltpu.VMEM((BPW, D), jnp.float32),
        pltpu.VMEM_SHARED((NS * BPW * L,), jnp.int32),
        pltpu.SMEM((CROWS * L,), jnp.int32),
        [pltpu.SemaphoreType.DMA] * NBUF,
    ],
)
def _avg_embed(idx_hbm, table_hbm, out_hbm, buf, out_v, idx_sh, idx_s, sems):
    sid = lax.axis_index("s")
    wid = sid * NC + lax.axis_index("c")
    base = wid * BPW

    pltpu.sync_copy(
        idx_hbm.at[pl.ds(wid * BPW * L, BPW * L)],
        idx_sh.at[pl.ds(sid * BPW * L, BPW * L)],
    )

    def fire(i, slot):
        for l in range(L):
            r = idx_s[i * L + l]
            pltpu.async_copy(
                table_hbm.at[pl.ds(r, 1), :],
                buf.at[slot, pl.ds(l, 1), :],
                sems[slot],
            )

    def drain(slot):
        for l in range(L):
            pltpu.make_async_copy(
                table_hbm.at[pl.ds(0, 1), :],
                buf.at[slot, pl.ds(l, 1), :],
                sems[slot],
            ).wait()

    def reduce(row, slot):
        acc0 = jnp.zeros((16,), jnp.float32)
        acc1 = jnp.zeros((16,), jnp.float32)
        for l in range(L):
            acc0 = acc0 + buf[slot, l, pl.ds(0, 16)]
            acc1 = acc1 + buf[slot, l, pl.ds(16, 16)]
        out_v[row, pl.ds(0, 16)] = acc0 * INV_L
        out_v[row, pl.ds(16, 16)] = acc1 * INV_L

    def chunk(blk):
        pltpu.sync_copy(
            idx_sh.at[pl.ds(sid * BPW * L + blk * CROWS * L, CROWS * L)], idx_s
        )
        for s in range(NBUF):
            fire(s, s)

        def rows(i):
            for s in range(NBUF):
                drain(s)

                @pl.when(i + s + NBUF < CROWS)
                def _():
                    fire(i + s + NBUF, s)

                reduce(blk * CROWS + i + s, s)

        pl.loop(0, CROWS, step=NBUF)(rows)

    pl.loop(0, NCHUNK)(chunk)

    pltpu.sync_copy(out_v, out_hbm.at[pl.ds(base, BPW), :])


def kernel(inputs, table):
    return _avg_embed(inputs.reshape(-1), table)
